# baseline (speedup 1.0000x reference)
import functools

import jax
import jax.numpy as jnp
from jax import lax
from jax.experimental import pallas as pl
from jax.experimental.pallas import tpu as pltpu
from jax.experimental.pallas import tpu_sc as plsc

VOCAB = 1000000
HIDDEN = 64
B, T = 1024, 200
N = B * T
NW = 32
RPW = N // NW
CH = 200
NCH = RPW // CH
GPC = CH // 16
C4 = HIDDEN // 16
PAIR = 128


def _sc_embed(tokens2, segments2, table2, pos_flat, seg_flat):
    mesh = plsc.VectorSubcoreMesh(core_axis_name="c", subcore_axis_name="s")

    @functools.partial(
        pl.kernel,
        mesh=mesh,
        out_type=jax.ShapeDtypeStruct((N * HIDDEN // 128, 128), jnp.float32),
        compiler_params=pltpu.CompilerParams(use_tc_tiling_on_sc=True),
        scratch_types=[
            pltpu.VMEM((RPW + 128,), jnp.int32),
            pltpu.VMEM((RPW + 128,), jnp.int32),
            pltpu.VMEM((CH * HIDDEN,), jnp.float32),
            pltpu.VMEM((2 * HIDDEN,), jnp.float32),
            pltpu.VMEM((2 * HIDDEN,), jnp.float32),
            pltpu.VMEM((2, CH, PAIR), jnp.float32),
            pltpu.VMEM((2, 2 * CH * HIDDEN // 128, 128), jnp.float32),
            pltpu.SemaphoreType.DMA,
            pltpu.SemaphoreType.DMA,
            pltpu.SemaphoreType.DMA,
        ],
    )
    def k(tok_hbm, seg_hbm, tt_hbm, pt_hbm, st_hbm, out_hbm,
          tokq, segq, ps0, seg_tab, dseg, tok_v, out_s, gsem0, gsem1, osem):
        w = lax.axis_index("s") * 2 + lax.axis_index("c")
        base = w * RPW
        gsems = (gsem0, gsem1)

        pltpu.sync_copy(tok_hbm.at[pl.ds(w * RPW, RPW)], tokq.at[pl.ds(0, RPW)])
        pltpu.sync_copy(seg_hbm.at[pl.ds(w * RPW, RPW)], segq.at[pl.ds(0, RPW)])
        pltpu.sync_copy(pt_hbm.at[pl.ds(0, T * HIDDEN)], ps0)
        pltpu.sync_copy(st_hbm, seg_tab)

        for c in range(C4):
            s0 = seg_tab[pl.ds(c * 16, 16)]
            dseg[pl.ds(c * 16, 16)] = seg_tab[pl.ds(HIDDEN + c * 16, 16)] - s0

        def ps0_body(r, carry):
            for c in range(C4):
                sl = pl.ds(r * HIDDEN + c * 16, 16)
                ps0[sl] = ps0[sl] + seg_tab[pl.ds(c * 16, 16)]
            return carry

        lax.fori_loop(0, CH, ps0_body, 0)

        def fire_gathers(ch, b):
            sem = gsems[b]

            def fg(g, carry):
                idxv = lax.shift_right_logical(
                    tokq[pl.ds(ch * CH + g * 16, 16)], 1)
                pltpu.async_copy(tt_hbm.at[idxv],
                                 tok_v.at[b, pl.ds(g * 16, 16)], sem)
                return carry

            lax.fori_loop(0, CH // 16, fg, 0)
            idxv = lax.shift_right_logical(
                tokq[pl.ds(ch * CH + CH - 16, 16)], 1)
            pltpu.async_copy(tt_hbm.at[idxv],
                             tok_v.at[b, pl.ds(CH - 16, 16)], gsems[b])

        def wait_gathers(ch, b):
            def wg(g, carry):
                pltpu.make_async_copy(
                    tt_hbm.at[pl.ds(0, 16)],
                    tok_v.at[b, pl.ds(g * 16, 16)], gsems[b]).wait()
                return carry

            lax.fori_loop(0, CH // 16, wg, 0)
            pltpu.make_async_copy(
                tt_hbm.at[pl.ds(0, 16)],
                tok_v.at[b, pl.ds(CH - 16, 16)], gsems[b]).wait()

        obase = w * (RPW * HIDDEN // 128)
        orows = 2 * CH * HIDDEN // 128

        def out_descr(i, b2):
            return pltpu.make_async_copy(
                out_s.at[b2], out_hbm.at[pl.ds(obase + i * orows, orows)], osem)

        dnums = lax.GatherDimensionNumbers(
            offset_dims=(), collapsed_slice_dims=(0,), start_index_map=(0,))
        dsegv = [dseg[pl.ds(c * 16, 16)] for c in range(C4)]

        def step(ch, b, b2):
            @pl.when(ch + 1 < NCH)
            def _():
                fire_gathers(ch + 1, 1 - b)

            wait_gathers(ch, b)

            def group_body(g, gc):
                off = ch * CH + g * 8
                segf = segq[pl.ds(off, 16)].astype(jnp.float32)
                hf = (tokq[pl.ds(off, 16)] & 1).astype(jnp.float32)
                for j in range(8):
                    sf = lax.gather(
                        segf, jnp.full((16, 1), j, jnp.int32), dnums,
                        slice_sizes=(1,),
                        mode=lax.GatherScatterMode.PROMISE_IN_BOUNDS)
                    hj = lax.gather(
                        hf, jnp.full((16, 1), j, jnp.int32), dnums,
                        slice_sizes=(1,),
                        mode=lax.GatherScatterMode.PROMISE_IN_BOUNDS)
                    r = g * 8 + j
                    for c in range(C4):
                        sl = pl.ds(c * 16, 16)
                        slh = pl.ds(HIDDEN + c * 16, 16)
                        flat = r * HIDDEN + c * 16
                        psl = pl.ds(flat, 16)
                        a = tok_v[b, r, sl]
                        bb = tok_v[b, r, slh]
                        tv = a + hj * (bb - a)
                        out_s[b2, b * (CH * HIDDEN // 128) + flat // 128,
                              pl.ds(flat % 128, 16)] = (
                            tv + ps0[psl] + sf * dsegv[c])
                return gc

            lax.fori_loop(0, CH // 8, group_body, 0)

        fire_gathers(0, 0)

        def pair_body(i, carry):
            b2 = i % 2

            @pl.when(i >= 2)
            def _():
                out_descr(i - 2, b2).wait()

            step(2 * i, 0, b2)
            step(2 * i + 1, 1, b2)
            pltpu.async_copy(out_s.at[b2],
                             out_hbm.at[pl.ds(obase + i * orows, orows)], osem)
            return carry

        lax.fori_loop(0, NCH // 2, pair_body, 0)
        out_descr(NCH // 2 - 2, (NCH // 2 - 2) % 2).wait()
        out_descr(NCH // 2 - 1, (NCH // 2 - 1) % 2).wait()

    return k(tokens2, segments2, table2, pos_flat, seg_flat)


def kernel(tokens, segments, token_table, pos_table, seg_table):
    tokens2 = tokens.astype(jnp.int32).reshape(-1)
    segments2 = segments.astype(jnp.int32).reshape(-1)
    table2 = token_table.reshape(VOCAB // 2, 2 * HIDDEN)
    out = _sc_embed(tokens2, segments2, table2,
                    pos_table.reshape(-1), seg_table.reshape(-1))
    return out.reshape(B, T, HIDDEN)


# speedup vs baseline: 1.2088x; 1.2088x over previous
import functools

import jax
import jax.numpy as jnp
from jax import lax
from jax.experimental import pallas as pl
from jax.experimental.pallas import tpu as pltpu
from jax.experimental.pallas import tpu_sc as plsc

VOCAB = 1000000
HIDDEN = 64
B, T = 1024, 200
N = B * T
NW = 32
RPW = N // NW
CH = 400
NCH = RPW // CH
GPC = CH // 16
C4 = HIDDEN // 16


def _sc_embed(tokens2, segments2, token_table, pos_flat, seg_flat):
    mesh = plsc.VectorSubcoreMesh(core_axis_name="c", subcore_axis_name="s")

    @functools.partial(
        pl.kernel,
        mesh=mesh,
        out_type=jax.ShapeDtypeStruct((N, HIDDEN), jnp.float32),
        compiler_params=pltpu.CompilerParams(use_tc_tiling_on_sc=False),
        scratch_types=[
            pltpu.VMEM((RPW,), jnp.int32),
            pltpu.VMEM((RPW,), jnp.int32),
            pltpu.VMEM((CH * HIDDEN,), jnp.float32),
            pltpu.VMEM((2 * HIDDEN,), jnp.float32),
            pltpu.VMEM((HIDDEN,), jnp.float32),
            pltpu.VMEM((2, CH, HIDDEN), jnp.float32),
            pltpu.SemaphoreType.DMA,
            pltpu.SemaphoreType.DMA,
            pltpu.SemaphoreType.DMA,
        ],
    )
    def k(tok_hbm, seg_hbm, tt_hbm, pt_hbm, st_hbm, out_hbm,
          tokq, segq, ps0, seg_tab, dseg, tok_v, gsem0, gsem1, osem):
        w = lax.axis_index("s") * 2 + lax.axis_index("c")
        base = w * RPW
        gsems = (gsem0, gsem1)

        pltpu.sync_copy(tok_hbm.at[w], tokq)
        pltpu.sync_copy(seg_hbm.at[w], segq)
        pltpu.sync_copy(pt_hbm.at[pl.ds(0, T * HIDDEN)], ps0.at[pl.ds(0, T * HIDDEN)])
        pltpu.sync_copy(pt_hbm.at[pl.ds(0, T * HIDDEN)],
                        ps0.at[pl.ds(T * HIDDEN, T * HIDDEN)])
        pltpu.sync_copy(st_hbm, seg_tab)

        for c in range(C4):
            s0 = seg_tab[pl.ds(c * 16, 16)]
            dseg[pl.ds(c * 16, 16)] = seg_tab[pl.ds(HIDDEN + c * 16, 16)] - s0

        def ps0_body(r, carry):
            for c in range(C4):
                sl = pl.ds(r * HIDDEN + c * 16, 16)
                ps0[sl] = ps0[sl] + seg_tab[pl.ds(c * 16, 16)]
            return carry

        lax.fori_loop(0, CH, ps0_body, 0)

        def fire_gathers(ch, b):
            sem = gsems[b]

            def fg(g, carry):
                idxv = tokq[pl.ds(ch * CH + g * 16, 16)]
                pltpu.async_copy(tt_hbm.at[idxv],
                                 tok_v.at[b, pl.ds(g * 16, 16)], sem)
                return carry

            lax.fori_loop(0, GPC, fg, 0)

        def wait_gathers(ch, b):
            pltpu.make_async_copy(
                tt_hbm.at[pl.ds(0, CH)], tok_v.at[b], gsems[b]).wait()

        def out_descr(ch, b):
            return pltpu.make_async_copy(
                tok_v.at[b], out_hbm.at[pl.ds(base + ch * CH, CH)], osem)

        dnums = lax.GatherDimensionNumbers(
            offset_dims=(), collapsed_slice_dims=(0,), start_index_map=(0,))
        dsegv = [dseg[pl.ds(c * 16, 16)] for c in range(C4)]

        def step(ch, b):
            @pl.when(ch >= 1)
            def _():
                out_descr(ch - 1, 1 - b).wait()

            @pl.when(ch + 1 < NCH)
            def _():
                fire_gathers(ch + 1, 1 - b)

            wait_gathers(ch, b)

            def group_body(g, gc):
                segf = segq[pl.ds(ch * CH + g * 16, 16)].astype(jnp.float32)
                for j in range(16):
                    sf = lax.gather(
                        segf, jnp.full((16, 1), j, jnp.int32), dnums,
                        slice_sizes=(1,),
                        mode=lax.GatherScatterMode.PROMISE_IN_BOUNDS)
                    r = g * 16 + j
                    for c in range(C4):
                        sl = pl.ds(c * 16, 16)
                        psl = pl.ds(r * HIDDEN + c * 16, 16)
                        tok_v[b, r, sl] = (tok_v[b, r, sl] + ps0[psl]
                                           + sf * dsegv[c])
                return gc

            lax.fori_loop(0, GPC, group_body, 0)

            pltpu.async_copy(tok_v.at[b],
                             out_hbm.at[pl.ds(base + ch * CH, CH)], osem)

        fire_gathers(0, 0)

        def pair_body(i, carry):
            step(2 * i, 0)
            step(2 * i + 1, 1)
            return carry

        lax.fori_loop(0, NCH // 2, pair_body, 0)
        out_descr(NCH - 1, (NCH - 1) % 2).wait()

    return k(tokens2, segments2, token_table, pos_flat, seg_flat)


def kernel(tokens, segments, token_table, pos_table, seg_table):
    tokens2 = tokens.astype(jnp.int32).reshape(NW, RPW)
    segments2 = segments.astype(jnp.int32).reshape(NW, RPW)
    out = _sc_embed(tokens2, segments2, token_table,
                    pos_table.reshape(-1), seg_table.reshape(-1))
    return out.reshape(B, T, HIDDEN)


# speedup vs baseline: 1.2122x; 1.0028x over previous
import functools

import jax
import jax.numpy as jnp
from jax import lax
from jax.experimental import pallas as pl
from jax.experimental.pallas import tpu as pltpu
from jax.experimental.pallas import tpu_sc as plsc

VOCAB = 1000000
HIDDEN = 64
B, T = 1024, 200
N = B * T
NW = 32
RPW = N // NW
CH = 400
NCH = RPW // CH
GPC = CH // 16
C4 = HIDDEN // 16


def _sc_embed(tokens2, segments2, token_table, pos_flat, seg_flat):
    mesh = plsc.VectorSubcoreMesh(core_axis_name="c", subcore_axis_name="s")

    @functools.partial(
        pl.kernel,
        mesh=mesh,
        out_type=jax.ShapeDtypeStruct((N, HIDDEN), jnp.float32),
        compiler_params=pltpu.CompilerParams(use_tc_tiling_on_sc=False),
        scratch_types=[
            pltpu.VMEM((RPW,), jnp.int32),
            pltpu.VMEM((RPW,), jnp.int32),
            pltpu.VMEM((CH * HIDDEN,), jnp.float32),
            pltpu.VMEM((2 * HIDDEN,), jnp.float32),
            pltpu.VMEM((HIDDEN,), jnp.float32),
            pltpu.VMEM((2, CH, HIDDEN), jnp.float32),
            pltpu.SemaphoreType.DMA,
            pltpu.SemaphoreType.DMA,
            pltpu.SemaphoreType.DMA,
        ],
    )
    def k(tok_hbm, seg_hbm, tt_hbm, pt_hbm, st_hbm, out_hbm,
          tokq, segq, ps0, seg_tab, dseg, tok_v, gsem0, gsem1, osem):
        w = lax.axis_index("s") * 2 + lax.axis_index("c")
        base = w * RPW
        gsems = (gsem0, gsem1)

        pltpu.sync_copy(tok_hbm.at[w], tokq)
        def fire_gathers(ch, b):
            sem = gsems[b]

            def fg(g, carry):
                idxv = tokq[pl.ds(ch * CH + g * 16, 16)]
                pltpu.async_copy(tt_hbm.at[idxv],
                                 tok_v.at[b, pl.ds(g * 16, 16)], sem)
                return carry

            lax.fori_loop(0, GPC, fg, 0)

        def wait_gathers(ch, b):
            pltpu.make_async_copy(
                tt_hbm.at[pl.ds(0, CH)], tok_v.at[b], gsems[b]).wait()

        def out_descr(ch, b):
            return pltpu.make_async_copy(
                tok_v.at[b], out_hbm.at[pl.ds(base + ch * CH, CH)], osem)

        fire_gathers(0, 0)
        pltpu.sync_copy(seg_hbm.at[w], segq)
        pltpu.sync_copy(pt_hbm.at[pl.ds(0, T * HIDDEN)], ps0.at[pl.ds(0, T * HIDDEN)])
        pltpu.sync_copy(pt_hbm.at[pl.ds(0, T * HIDDEN)],
                        ps0.at[pl.ds(T * HIDDEN, T * HIDDEN)])
        pltpu.sync_copy(st_hbm, seg_tab)

        for c in range(C4):
            s0 = seg_tab[pl.ds(c * 16, 16)]
            dseg[pl.ds(c * 16, 16)] = seg_tab[pl.ds(HIDDEN + c * 16, 16)] - s0

        def ps0_body(r, carry):
            for c in range(C4):
                sl = pl.ds(r * HIDDEN + c * 16, 16)
                ps0[sl] = ps0[sl] + seg_tab[pl.ds(c * 16, 16)]
            return carry

        lax.fori_loop(0, CH, ps0_body, 0)

        dnums = lax.GatherDimensionNumbers(
            offset_dims=(), collapsed_slice_dims=(0,), start_index_map=(0,))
        dsegv = [dseg[pl.ds(c * 16, 16)] for c in range(C4)]

        def step(ch, b):
            @pl.when(ch >= 1)
            def _():
                out_descr(ch - 1, 1 - b).wait()

            @pl.when(ch + 1 < NCH)
            def _():
                fire_gathers(ch + 1, 1 - b)

            wait_gathers(ch, b)

            def group_body(g, gc):
                segf = segq[pl.ds(ch * CH + g * 16, 16)].astype(jnp.float32)
                for j in range(16):
                    sf = lax.gather(
                        segf, jnp.full((16, 1), j, jnp.int32), dnums,
                        slice_sizes=(1,),
                        mode=lax.GatherScatterMode.PROMISE_IN_BOUNDS)
                    r = g * 16 + j
                    for c in range(C4):
                        sl = pl.ds(c * 16, 16)
                        psl = pl.ds(r * HIDDEN + c * 16, 16)
                        tok_v[b, r, sl] = (tok_v[b, r, sl] + ps0[psl]
                                           + sf * dsegv[c])
                return gc

            lax.fori_loop(0, GPC, group_body, 0)

            pltpu.async_copy(tok_v.at[b],
                             out_hbm.at[pl.ds(base + ch * CH, CH)], osem)

        def pair_body(i, carry):
            step(2 * i, 0)
            step(2 * i + 1, 1)
            return carry

        lax.fori_loop(0, NCH // 2, pair_body, 0)
        out_descr(NCH - 1, (NCH - 1) % 2).wait()

    return k(tokens2, segments2, token_table, pos_flat, seg_flat)


def kernel(tokens, segments, token_table, pos_table, seg_table):
    tokens2 = tokens.astype(jnp.int32).reshape(NW, RPW)
    segments2 = segments.astype(jnp.int32).reshape(NW, RPW)
    out = _sc_embed(tokens2, segments2, token_table,
                    pos_table.reshape(-1), seg_table.reshape(-1))
    return out.reshape(B, T, HIDDEN)
